# Initial kernel scaffold; baseline (speedup 1.0000x reference)
import jax
import jax.numpy as jnp
from jax.experimental import pallas as pl


def kernel(x, weights, idx_a, idx_b):
    raise NotImplementedError("write your pallas kernel here")


# speedup vs baseline: 2.1501x; 2.1501x over previous
import functools

import jax
import jax.numpy as jnp
from jax import lax
from jax.experimental import pallas as pl
from jax.experimental.pallas import tpu as pltpu
from jax.experimental.pallas import tpu_sc as plsc

L = 16
NCORE = 2
NSUB = 16
NW = NCORE * NSUB
BATCH = 128
NK = 32
OH = OW = 30
P = OH * OW
PCH = (P + L - 1) // L
XW = 3 * 32 * 32
XPAD = 3104
OUTW = NK * P
OUTPAD = OUTW + 16
B_PER_W = BATCH // NW


def _tec_body(x_hbm, w_hbm, ia_hbm, ib_hbm, out_hbm,
              x_v, w_v, idx_v, pos_v, coef_v, off_v, out_v):
    wid = lax.axis_index("s") * NCORE + lax.axis_index("c")

    pltpu.sync_copy(w_hbm, w_v)
    pltpu.sync_copy(ia_hbm, idx_v.at[pl.ds(0, NK)])
    pltpu.sync_copy(ib_hbm, idx_v.at[pl.ds(NK, NK)])

    lanes = lax.broadcasted_iota(jnp.int32, (L,), 0)

    def vdiv(v, d):
        return lax.div(v, jnp.full((L,), d, jnp.int32))

    def pos_body(pch, _):
        p = lanes + pch * L
        i = vdiv(p, OW)
        pos_v[pl.ds(pch * L, L)] = i * 32 + (p - i * OW)
        return 0
    lax.fori_loop(0, PCH, pos_body, 0)

    def off_body(q, _):
        iv = idx_v[pl.ds(q * L, L)]
        c = vdiv(iv, 9)
        r = iv - 9 * c
        h = vdiv(r, 3)
        off_v[pl.ds(q * L, L)] = c * 1024 + h * 32 + (r - 3 * h)
        return 0
    lax.fori_loop(0, (2 * NK) // L, off_body, 0)

    for q in range(NK // L):
        base = lanes * L + q * L * L
        z = jnp.zeros((L,), jnp.float32)
        acc = [z, z, z, z]
        for i in range(1, 16):
            col = plsc.load_gather(w_v, [base + i])
            if i & 8:
                acc[0] = acc[0] + col
            if i & 4:
                acc[1] = acc[1] + col
            if i & 2:
                acc[2] = acc[2] + col
            if i & 1:
                acc[3] = acc[3] + col
        c00, c01, c10, c11 = acc
        coef_v[pl.ds(q * L, L)] = c00
        coef_v[pl.ds(NK + q * L, L)] = c10 - c00
        coef_v[pl.ds(2 * NK + q * L, L)] = c01 - c00
        coef_v[pl.ds(3 * NK + q * L, L)] = c11 - c10 - c01 + c00

    def batch_body(bi, _):
        b = wid * B_PER_W + bi
        pltpu.sync_copy(x_hbm.at[b], x_v.at[pl.ds(0, XW)])

        def k_body(k, _):
            kv = jnp.full((L,), k, jnp.int32)
            c0 = plsc.load_gather(coef_v, [kv])
            ca = plsc.load_gather(coef_v, [kv + NK])
            cb = plsc.load_gather(coef_v, [kv + 2 * NK])
            cab = plsc.load_gather(coef_v, [kv + 3 * NK])
            offa = plsc.load_gather(off_v, [kv])
            offb = plsc.load_gather(off_v, [kv + NK])
            obase = k * P

            def p_body(pch, _):
                pos = pos_v[pl.ds(pch * L, L)]
                a = plsc.load_gather(x_v, [pos + offa])
                bb = plsc.load_gather(x_v, [pos + offb])
                out_v[pl.ds(obase + pch * L, L)] = (
                    c0 + ca * a + cb * bb + cab * (a * bb))
                return 0
            lax.fori_loop(0, PCH, p_body, 0)
            return 0
        lax.fori_loop(0, NK, k_body, 0)

        pltpu.sync_copy(out_v.at[pl.ds(0, OUTW)], out_hbm.at[b])
        return 0
    lax.fori_loop(0, B_PER_W, batch_body, 0)


_sc_call = functools.partial(
    pl.kernel,
    out_type=jax.ShapeDtypeStruct((BATCH, OUTW), jnp.float32),
    mesh=plsc.VectorSubcoreMesh(
        core_axis_name="c", subcore_axis_name="s",
        num_cores=NCORE, num_subcores=NSUB),
    compiler_params=pltpu.CompilerParams(needs_layout_passes=False),
    scratch_types=[
        pltpu.VMEM((XPAD,), jnp.float32),
        pltpu.VMEM((NK * 16,), jnp.float32),
        pltpu.VMEM((2 * NK,), jnp.int32),
        pltpu.VMEM((PCH * L,), jnp.int32),
        pltpu.VMEM((4 * NK,), jnp.float32),
        pltpu.VMEM((2 * NK,), jnp.int32),
        pltpu.VMEM((OUTPAD,), jnp.float32),
    ],
)(_tec_body)


@jax.jit
def kernel(x, weights, idx_a, idx_b):
    xf = x.reshape(BATCH, XW)
    wf = weights.reshape(NK * 16)
    out = _sc_call(xf, wf, idx_a.astype(jnp.int32), idx_b.astype(jnp.int32))
    return out.reshape(BATCH, NK, OH, OW)
